# baseline (speedup 1.0000x reference)
import functools

import jax
import jax.numpy as jnp
from jax import lax
from jax.experimental import pallas as pl
from jax.experimental.pallas import tpu as pltpu
from jax.experimental.pallas import tpu_sc as plsc

N = 50000
E = 800000
D = 64
R = 32
CUTOFF = 5.0

NC = 2
NS = 16
CHUNK = 32
N_PAD = 50176
ROWS_PT = N_PAD // NS
NCH_N = ROWS_PT // CHUNK
E_PAD = 802816
ECH_PT = E_PAD // NS // CHUNK
ECH_PW = E_PAD // (NS * NC) // CHUNK
HALF = 32

_mesh = plsc.VectorSubcoreMesh(core_axis_name="c", subcore_axis_name="s")


def _zero_rows(ref, ncols):
    zero16 = jnp.zeros((16,), jnp.float32)

    @pl.loop(0, CHUNK)
    def _(r):
        for o in range(0, ncols, 16):
            ref[r, pl.ds(o, 16)] = zero16


def _half_row(buf, c, i, o):
    return jnp.where(c == 0, buf[i, pl.ds(o, 16)], buf[i, pl.ds(32 + o, 16)])


def _emb_body(tab, z_hbm, zr_hbm, olig, osig, ores,
              idx_v, adj_v, rows_v, sem):
    s = lax.axis_index("s")
    c = lax.axis_index("c")

    @pl.when(c == 0)
    def _():
        @pl.loop(0, NCH_N)
        def _(j):
            base = s * ROWS_PT + j * CHUNK
            pltpu.sync_copy(z_hbm.at[pl.ds(base, CHUNK)], idx_v)
            for off, out in ((0, olig), (119, osig)):
                for i in range(CHUNK // 16):
                    adj_v[pl.ds(i * 16, 16)] = idx_v[pl.ds(i * 16, 16)] + off
                pltpu.async_copy(tab.at[adj_v], rows_v, sem).wait()
                pltpu.sync_copy(rows_v, out.at[pl.ds(base, CHUNK)])

    @pl.when(c == 1)
    def _():
        @pl.loop(0, NCH_N)
        def _(j):
            base = s * ROWS_PT + j * CHUNK
            pltpu.sync_copy(zr_hbm.at[pl.ds(base, CHUNK)], idx_v)
            for i in range(CHUNK // 16):
                adj_v[pl.ds(i * 16, 16)] = idx_v[pl.ds(i * 16, 16)] + 238
            pltpu.async_copy(tab.at[adj_v], rows_v, sem).wait()
            pltpu.sync_copy(rows_v, ores.at[pl.ds(base, CHUNK)])


def _prep_emb(tab, z_pad, zr_pad):
    f = pl.kernel(
        _emb_body,
        out_type=[jax.ShapeDtypeStruct((N_PAD, 128), jnp.float32)] * 3,
        mesh=_mesh,
        scratch_types=[
            pltpu.VMEM((CHUNK,), jnp.int32),
            pltpu.VMEM((CHUNK,), jnp.int32),
            pltpu.VMEM((CHUNK, 128), jnp.float32),
            pltpu.SemaphoreType.DMA,
        ],
    )
    return f(tab, z_pad, zr_pad)


def _diff_body(offA, offB, ia_hbm, ib_hbm, postab, diff_hbm,
               ia_v, ib_v, pa_v, pb_v, d16_v, semA, semB):
    c = lax.axis_index("c")
    s = lax.axis_index("s")
    w = s * NC + c

    @pl.loop(0, ECH_PW)
    def _(k):
        base = w * (ECH_PW * CHUNK) + k * CHUNK
        pltpu.sync_copy(ia_hbm.at[pl.ds(base, CHUNK)], ia_v)
        pltpu.sync_copy(ib_hbm.at[pl.ds(base, CHUNK)], ib_v)
        for i in range(CHUNK // 16):
            if offA:
                ia_v[pl.ds(i * 16, 16)] = ia_v[pl.ds(i * 16, 16)] + offA
            if offB:
                ib_v[pl.ds(i * 16, 16)] = ib_v[pl.ds(i * 16, 16)] + offB
        cpA = pltpu.async_copy(postab.at[ia_v], pa_v, semA)
        cpB = pltpu.async_copy(postab.at[ib_v], pb_v, semB)
        cpA.wait()
        cpB.wait()

        @pl.loop(0, CHUNK)
        def _(r):
            d16_v[r, pl.ds(0, 16)] = (pb_v[r, pl.ds(0, 16)]
                                      - pa_v[r, pl.ds(0, 16)])

        pltpu.sync_copy(d16_v, diff_hbm.at[pl.ds(base, CHUNK)])


def _prep_diff(offA, offB):
    return pl.kernel(
        functools.partial(_diff_body, offA, offB),
        out_type=jax.ShapeDtypeStruct((E_PAD, 16), jnp.float32),
        mesh=_mesh,
        scratch_types=[
            pltpu.VMEM((CHUNK,), jnp.int32),
            pltpu.VMEM((CHUNK,), jnp.int32),
            pltpu.VMEM((CHUNK, 128), jnp.float32),
            pltpu.VMEM((CHUNK, 128), jnp.float32),
            pltpu.VMEM((CHUNK, 16), jnp.float32),
            pltpu.SemaphoreType.DMA,
            pltpu.SemaphoreType.DMA,
        ],
    )


def _agg_body(two_src, sub, refs):
    if two_src:
        (srcA_hbm, srcB_hbm, dst_hbm, htabA, htabB, rw_hbm,
         msg_hbm, idxA_v, idxB_v, dst_v, rowsA_v, rowsB_v, rw_v,
         msg_v, semA, semB) = refs
    else:
        (srcA_hbm, dst_hbm, htabA, rw_hbm,
         msg_hbm, idxA_v, idxB_v, dst_v, rowsA_v, rowsB_v, rw_v,
         msg_v, semA, semB) = refs

    c = lax.axis_index("c")
    s = lax.axis_index("s")

    @pl.loop(0, ECH_PT)
    def _(k):
        base = s * (ECH_PT * CHUNK) + k * CHUNK
        pltpu.sync_copy(srcA_hbm.at[pl.ds(base, CHUNK)], idxA_v)
        cpA = pltpu.async_copy(htabA.at[idxA_v], rowsA_v, semA)
        if two_src:
            pltpu.sync_copy(srcB_hbm.at[pl.ds(base, CHUNK)], idxB_v)
            cpB = pltpu.async_copy(htabB.at[idxB_v], rowsB_v, semB)
        pltpu.sync_copy(rw_hbm.at[pl.ds(base, CHUNK)], rw_v)
        cpA.wait()
        if two_src:
            cpB.wait()
        for i in range(CHUNK):
            for o in (0, 16):
                m = (_half_row(rowsA_v, c, i, o)
                     * _half_row(rw_v, c, i, sub + o))
                if two_src:
                    m = m * _half_row(rowsB_v, c, i, o)
                msg_v[i, pl.ds(o, 16)] = m
        pltpu.sync_copy(msg_v, msg_hbm.at[pl.ds(c * E_PAD + base, CHUNK)])


def _edge_agg(two_src, sub):
    scratch = [
        pltpu.VMEM((CHUNK,), jnp.int32),
        pltpu.VMEM((CHUNK,), jnp.int32),
        pltpu.VMEM((CHUNK,), jnp.int32),
        pltpu.VMEM((CHUNK, 128), jnp.float32),
        pltpu.VMEM((CHUNK, 128), jnp.float32),
        pltpu.VMEM((CHUNK, 128), jnp.float32),
        pltpu.VMEM((CHUNK, HALF), jnp.float32),
        pltpu.SemaphoreType.DMA,
        pltpu.SemaphoreType.DMA,
    ]

    def body(*refs):
        _agg_body(two_src, sub, refs)

    return pl.kernel(body,
                     out_type=jax.ShapeDtypeStruct((NC * E_PAD, HALF),
                                                   jnp.float32),
                     mesh=_mesh, scratch_types=scratch)


def _vec_body(two_src, refs):
    if two_src:
        (srcA_hbm, srcB_hbm, dst_hbm, htabA, htabB, rwv_hbm,
         sv_hbm, idxA_v, idxB_v, dst_v, rowsA_v, rowsB_v, rwv_v,
         sv_v, semA, semB) = refs
    else:
        (srcA_hbm, dst_hbm, htabA, rwv_hbm,
         sv_hbm, idxA_v, idxB_v, dst_v, rowsA_v, rowsB_v, rwv_v,
         sv_v, semA, semB) = refs

    c = lax.axis_index("c")
    s = lax.axis_index("s")

    _zero_rows(sv_v, HALF)

    @pl.loop(0, ECH_PT)
    def _(k):
        base = s * (ECH_PT * CHUNK) + k * CHUNK
        pltpu.sync_copy(srcA_hbm.at[pl.ds(base, CHUNK)], idxA_v)
        cpA = pltpu.async_copy(htabA.at[idxA_v], rowsA_v, semA)
        if two_src:
            pltpu.sync_copy(srcB_hbm.at[pl.ds(base, CHUNK)], idxB_v)
            cpB = pltpu.async_copy(htabB.at[idxB_v], rowsB_v, semB)
        pltpu.sync_copy(rwv_hbm.at[pl.ds(base, CHUNK)], rwv_v)
        cpA.wait()
        if two_src:
            cpB.wait()
        for i in range(CHUNK):
            if two_src:
                t0 = (_half_row(rowsA_v, c, i, 0) * _half_row(rowsB_v, c, i, 0)
                      * _half_row(rwv_v, c, i, 0))
                t1 = (_half_row(rowsA_v, c, i, 16)
                      * _half_row(rowsB_v, c, i, 16)
                      * _half_row(rwv_v, c, i, 16))
            else:
                t0 = _half_row(rowsA_v, c, i, 0) * _half_row(rwv_v, c, i, 0)
                t1 = _half_row(rowsA_v, c, i, 16) * _half_row(rwv_v, c, i, 16)
            v = t0 + t1
            for sh in (8, 4, 2, 1):
                sv_v[i, pl.ds(0, 16)] = v
                v = v + sv_v[i, pl.ds(sh, 16)]
            sc = v[0]
            sv_v[i, pl.ds(0, 16)] = rwv_v[i, pl.ds(64, 16)] * sc
        pltpu.sync_copy(sv_v, sv_hbm.at[pl.ds(c * E_PAD + base, CHUNK)])


def _edge_vec(two_src):
    scratch = [
        pltpu.VMEM((CHUNK,), jnp.int32),
        pltpu.VMEM((CHUNK,), jnp.int32),
        pltpu.VMEM((CHUNK,), jnp.int32),
        pltpu.VMEM((CHUNK, 128), jnp.float32),
        pltpu.VMEM((CHUNK, 128), jnp.float32),
        pltpu.VMEM((CHUNK, 128), jnp.float32),
        pltpu.VMEM((CHUNK, HALF), jnp.float32),
        pltpu.SemaphoreType.DMA,
        pltpu.SemaphoreType.DMA,
    ]

    def body(*refs):
        _vec_body(two_src, refs)

    return pl.kernel(body,
                     out_type=jax.ShapeDtypeStruct((NC * E_PAD, HALF),
                                                   jnp.float32),
                     mesh=_mesh, scratch_types=scratch)


def _rw_body(with_unit, diff_ref, wr_ref, rw_ref):
    i = pl.program_id(0)
    blk = diff_ref.shape[0]
    diff = diff_ref[...]
    x, y, zz = diff[:, 0:1], diff[:, 1:2], diff[:, 2:3]
    d2 = x * x + y * y + zz * zz
    d = jnp.sqrt(d2 + 1e-8)
    row = i * blk + lax.broadcasted_iota(jnp.int32, (blk, 1), 0)
    mask = row < E
    centers = (lax.broadcasted_iota(jnp.int32, (1, R), 1).astype(jnp.float32)
               * (CUTOFF / (R - 1)))
    g = R / CUTOFF
    r = jnp.where(mask, jnp.exp(-g * (d - centers) ** 2), 0.0)
    out = jnp.dot(r, wr_ref[0], preferred_element_type=jnp.float32)
    if with_unit:
        unit = jnp.where(mask, diff / d, 0.0)
        col = lax.broadcasted_iota(jnp.int32, (blk, 128), 1)
        unit128 = jnp.pad(unit, ((0, 0), (64, 48)))
        out = jnp.where((col >= 64) & (col < 80), unit128, out)
    rw_ref[0] = out


def _rw_tc(diff16, wr_stack, with_unit):
    nl = wr_stack.shape[0]
    blk = 2048
    grid = (E_PAD // blk, nl)
    return pl.pallas_call(
        functools.partial(_rw_body, with_unit),
        grid=grid,
        in_specs=[
            pl.BlockSpec((blk, 16), lambda i, j: (i, 0)),
            pl.BlockSpec((1, R, 128), lambda i, j: (j, 0, 0)),
        ],
        out_specs=pl.BlockSpec((1, blk, 128), lambda i, j: (j, i, 0)),
        out_shape=jax.ShapeDtypeStruct((nl, E_PAD, 128), jnp.float32),
    )(diff16, wr_stack)


def _node_body(h_ref, agg_ref, wh_ref, out_ref):
    h = h_ref[...]
    up = jnp.tanh(jnp.dot(agg_ref[...], wh_ref[...],
                          preferred_element_type=jnp.float32))
    out_ref[...] = h + jnp.pad(up, ((0, 0), (0, 64)))


def _node_update(h, agg, wh):
    blk = 1024
    return pl.pallas_call(
        _node_body,
        grid=(N_PAD // blk,),
        in_specs=[
            pl.BlockSpec((blk, 128), lambda i: (i, 0)),
            pl.BlockSpec((blk, D), lambda i: (i, 0)),
            pl.BlockSpec((D, D), lambda i: (0, 0)),
        ],
        out_specs=pl.BlockSpec((blk, 128), lambda i: (i, 0)),
        out_shape=jax.ShapeDtypeStruct((N_PAD, 128), jnp.float32),
    )(h, agg, wh)


def _geom_body(h_ref, pn_ref, pca_ref, pc_ref, wg_ref, out_ref):
    v1 = pn_ref[...] - pca_ref[...]
    v2 = pc_ref[...] - pca_ref[...]
    n1 = jnp.sqrt(jnp.sum(v1[:, :3] * v1[:, :3], axis=1, keepdims=True) + 1e-8)
    n2 = jnp.sqrt(jnp.sum(v2[:, :3] * v2[:, :3], axis=1, keepdims=True) + 1e-8)
    cos = jnp.sum(v1[:, :3] * v2[:, :3], axis=1, keepdims=True) / (n1 * n2)
    geom = jnp.concatenate([n1, n2, cos, jnp.zeros_like(n1)], axis=1)
    up = jnp.tanh(jnp.dot(geom, wg_ref[...],
                          preferred_element_type=jnp.float32))
    out_ref[...] = h_ref[...] + jnp.pad(up, ((0, 0), (0, 64)))


def _geom_tc(h0, pn4, pca4, pc4, wg4):
    blk = 1024
    return pl.pallas_call(
        _geom_body,
        grid=(N_PAD // blk,),
        in_specs=[
            pl.BlockSpec((blk, 128), lambda i: (i, 0)),
            pl.BlockSpec((blk, 4), lambda i: (i, 0)),
            pl.BlockSpec((blk, 4), lambda i: (i, 0)),
            pl.BlockSpec((blk, 4), lambda i: (i, 0)),
            pl.BlockSpec((4, D), lambda i: (0, 0)),
        ],
        out_specs=pl.BlockSpec((blk, 128), lambda i: (i, 0)),
        out_shape=jax.ShapeDtypeStruct((N_PAD, 128), jnp.float32),
    )(h0, pn4, pca4, pc4, wg4)


def _final_body(v0_ref, v1_ref, v2_ref, v3_ref, hs_ref, ws_ref, noise_ref,
                mass_ref, gamma_ref, out_ref):
    vec = v0_ref[...] + v1_ref[...] + v2_ref[...] + v3_ref[...]
    h64 = hs_ref[...][:, :D]
    logits = jnp.dot(h64, ws_ref[...], preferred_element_type=jnp.float32)
    sigma = jnp.logaddexp(logits, 0.0)
    F = vec + gamma_ref[...] * noise_ref[...] * sigma
    out_ref[...] = F / mass_ref[...]


def _final_tc(vecs, h_sig, ws, noise, mass, gamma32):
    blk = 1024
    vecs3 = vecs
    return pl.pallas_call(
        _final_body,
        grid=(N_PAD // blk,),
        in_specs=[
            pl.BlockSpec((blk, HALF), lambda i: (i, 0)),
            pl.BlockSpec((blk, HALF), lambda i: (i, 0)),
            pl.BlockSpec((blk, HALF), lambda i: (i, 0)),
            pl.BlockSpec((blk, HALF), lambda i: (i, 0)),
            pl.BlockSpec((blk, 128), lambda i: (i, 0)),
            pl.BlockSpec((D, 1), lambda i: (0, 0)),
            pl.BlockSpec((blk, 1), lambda i: (i, 0)),
            pl.BlockSpec((blk, 1), lambda i: (i, 0)),
            pl.BlockSpec((1, HALF), lambda i: (0, 0)),
        ],
        out_specs=pl.BlockSpec((blk, HALF), lambda i: (i, 0)),
        out_shape=jax.ShapeDtypeStruct((N_PAD, HALF), jnp.float32),
    )(*vecs3, h_sig, ws, noise, mass, gamma32)


def _tab128(w):
    return jnp.pad(w, ((0, 0), (0, 64)))


def _pad_e(a, fill=0):
    return jnp.pad(a, (0, E_PAD - E), constant_values=fill)


def _pad128(p):
    return jnp.pad(p, ((0, 0), (0, 125)))


def kernel(t, velocity, ligand_positions, z, batch, ligand_mass, pos_N, pos_Ca,
           pos_C, residue_type, batch_residue, edge_index_ligand,
           edge_index_residue, edge_index_complex, params):
    pl_, ps_, pp_, pc_ = (params['lig'], params['sig'], params['prot'],
                          params['cpx'])

    tab = jnp.concatenate([_tab128(pl_['emb']), _tab128(ps_['emb']),
                           _tab128(pp_['res_emb'])], axis=0)
    z_pad = jnp.pad(z.astype(jnp.int32), (0, N_PAD - N))
    zr_pad = jnp.pad(residue_type.astype(jnp.int32), (0, N_PAD - N))
    postab = jnp.concatenate([_pad128(ligand_positions), _pad128(pos_Ca)], 0)

    srcl = _pad_e(edge_index_ligand[0].astype(jnp.int32))
    dstl = _pad_e(edge_index_ligand[1].astype(jnp.int32))
    srcr = _pad_e(edge_index_residue[0].astype(jnp.int32))
    dstr = _pad_e(edge_index_residue[1].astype(jnp.int32))
    ligc = _pad_e(edge_index_complex[0].astype(jnp.int32))
    resc = _pad_e(edge_index_complex[1].astype(jnp.int32))

    wr_lig = jnp.stack([jnp.concatenate([pl_['Wr0'], pl_['Wr1']], axis=1),
                        jnp.concatenate([ps_['Wr0'], ps_['Wr1']], axis=1)])
    wr_res = jnp.stack([jnp.pad(pp_['Wr'], ((0, 0), (0, 64)))])
    wr_cpx = jnp.stack([jnp.concatenate([pc_['Wr0'], pc_['Wr1']], axis=1)])
    wv_lig = jnp.stack([
        jnp.pad(pl_['Wr0'] * pl_['wv0'][None, :], ((0, 0), (0, 64))),
        jnp.pad(pl_['Wr1'] * pl_['wv1'][None, :], ((0, 0), (0, 64)))])
    wv_cpx = jnp.stack([
        jnp.pad(pc_['Wr0'] * pc_['wv0'][None, :], ((0, 0), (0, 64))),
        jnp.pad(pc_['Wr1'] * pc_['wv1'][None, :], ((0, 0), (0, 64)))])

    h_lig, h_sig, h_res = _prep_emb(tab, z_pad, zr_pad)
    diff_l = _prep_diff(0, 0)(srcl, dstl, postab)
    diff_r = _prep_diff(N, N)(srcr, dstr, postab)
    diff_c = _prep_diff(0, N)(ligc, resc, postab)

    rw_l = _rw_tc(diff_l, wr_lig, False)
    rw_r = _rw_tc(diff_r, wr_res, False)
    rw_c = _rw_tc(diff_c, wr_cpx, False)
    rwv_l = _rw_tc(diff_l, wv_lig, True)
    rwv_c = _rw_tc(diff_c, wv_cpx, True)

    wg4 = jnp.pad(pp_['Wg'], ((0, 1), (0, 0)))
    pad_n = ((0, N_PAD - N), (0, 0))
    pn4 = jnp.pad(jnp.pad(pos_N, ((0, 0), (0, 1))), pad_n)
    pca4 = jnp.pad(jnp.pad(pos_Ca, ((0, 0), (0, 1))), pad_n)
    pc4 = jnp.pad(jnp.pad(pos_C, ((0, 0), (0, 1))), pad_n)
    h_res = _geom_tc(h_res, pn4, pca4, pc4, wg4)

    aggA1 = _edge_agg(False, 0)
    aggB1 = _edge_agg(False, 64)
    aggA2 = _edge_agg(True, 0)
    vec1 = _edge_vec(False)
    vec2 = _edge_vec(True)

    def seg(rows, idx):
        m = rows.reshape(NC, E_PAD, HALF)
        return jax.ops.segment_sum(jnp.concatenate([m[0], m[1]], 1), idx,
                                   num_segments=N_PAD)

    def seg_sv(rows, idx):
        m = rows.reshape(NC, E_PAD, HALF)
        return jax.ops.segment_sum(m[0] + m[1], idx, num_segments=N_PAD)

    vecs = []
    h = h_lig
    for l in range(2):
        msg = (aggA1 if l == 0 else aggB1)(srcl, dstl, h, rw_l[0])
        vecs.append(seg_sv(vec1(srcl, dstl, h, rwv_l[l]), dstl))
        h = _node_update(h, seg(msg, dstl), pl_['Wh%d' % l])
    ligand_repr = h

    h = h_sig
    for l in range(2):
        msg = (aggA1 if l == 0 else aggB1)(srcl, dstl, h, rw_l[1])
        h = _node_update(h, seg(msg, dstl), ps_['Wh%d' % l])
    h_sig_f = h

    msg = aggA1(srcr, dstr, h_res, rw_r[0])
    h_res = _node_update(h_res, seg(msg, dstr), pp_['Wh'])

    h = ligand_repr
    for l in range(2):
        vecs.append(seg_sv(vec2(ligc, resc, ligc, h, h_res, rwv_c[l]), ligc))
        if l == 0:
            msg = aggA2(ligc, resc, ligc, h, h_res, rw_c[0])
            h = _node_update(h, seg(msg, ligc), pc_['Wh0'])

    noise = jax.random.normal(jax.random.key(42), (N, 1))
    noise = jnp.pad(noise, ((0, N_PAD - N), (0, 0)))
    mass = jnp.pad(ligand_mass.reshape(N, 1), ((0, N_PAD - N), (0, 0)),
                   constant_values=1.0)
    gamma32 = jnp.pad(params['gamma'], ((0, 0), (0, 29)))
    out = _final_tc(vecs, h_sig_f, params['w_sigma'], noise, mass, gamma32)
    return (out[:N, :3], velocity)


# speedup vs baseline: 1.2367x; 1.2367x over previous
import functools

import jax
import jax.numpy as jnp
from jax import lax
from jax.experimental import pallas as pl
from jax.experimental.pallas import tpu as pltpu
from jax.experimental.pallas import tpu_sc as plsc

N = 50000
E = 800000
D = 64
R = 32
CUTOFF = 5.0

NC = 2
NS = 16
CHUNK = 128
N_PAD = 51200
ROWS_PT = N_PAD // NS
NCH_N = ROWS_PT // CHUNK
E_PAD = 802816
ECH_PT = E_PAD // NS // CHUNK
ECH_PW = E_PAD // (NS * NC) // CHUNK
HALF = 32

_mesh = plsc.VectorSubcoreMesh(core_axis_name="c", subcore_axis_name="s")


def _zero_rows(ref, ncols):
    zero16 = jnp.zeros((16,), jnp.float32)

    @pl.loop(0, CHUNK)
    def _(r):
        for o in range(0, ncols, 16):
            ref[r, pl.ds(o, 16)] = zero16


def _half_row(buf, c, i, o):
    return jnp.where(c == 0, buf[i, pl.ds(o, 16)], buf[i, pl.ds(32 + o, 16)])


def _emb_body(tab, z_hbm, zr_hbm, olig, osig, ores,
              idx_v, adj_v, rows_v, sem):
    s = lax.axis_index("s")
    c = lax.axis_index("c")

    @pl.when(c == 0)
    def _():
        @pl.loop(0, NCH_N)
        def _(j):
            base = s * ROWS_PT + j * CHUNK
            pltpu.sync_copy(z_hbm.at[pl.ds(base, CHUNK)], idx_v)
            for off, out in ((0, olig), (119, osig)):
                for i in range(CHUNK // 16):
                    adj_v[pl.ds(i * 16, 16)] = idx_v[pl.ds(i * 16, 16)] + off
                pltpu.async_copy(tab.at[adj_v], rows_v, sem).wait()
                pltpu.sync_copy(rows_v, out.at[pl.ds(base, CHUNK)])

    @pl.when(c == 1)
    def _():
        @pl.loop(0, NCH_N)
        def _(j):
            base = s * ROWS_PT + j * CHUNK
            pltpu.sync_copy(zr_hbm.at[pl.ds(base, CHUNK)], idx_v)
            for i in range(CHUNK // 16):
                adj_v[pl.ds(i * 16, 16)] = idx_v[pl.ds(i * 16, 16)] + 238
            pltpu.async_copy(tab.at[adj_v], rows_v, sem).wait()
            pltpu.sync_copy(rows_v, ores.at[pl.ds(base, CHUNK)])


def _prep_emb(tab, z_pad, zr_pad):
    f = pl.kernel(
        _emb_body,
        out_type=[jax.ShapeDtypeStruct((N_PAD, 128), jnp.float32)] * 3,
        mesh=_mesh,
        scratch_types=[
            pltpu.VMEM((CHUNK,), jnp.int32),
            pltpu.VMEM((CHUNK,), jnp.int32),
            pltpu.VMEM((CHUNK, 128), jnp.float32),
            pltpu.SemaphoreType.DMA,
        ],
    )
    return f(tab, z_pad, zr_pad)


def _diff_body(offA, offB, ia_hbm, ib_hbm, postab, diff_hbm,
               ia_v, ib_v, pa_v, pb_v, d16_v, semA, semB):
    c = lax.axis_index("c")
    s = lax.axis_index("s")
    w = s * NC + c

    @pl.loop(0, ECH_PW)
    def _(k):
        base = w * (ECH_PW * CHUNK) + k * CHUNK
        pltpu.sync_copy(ia_hbm.at[pl.ds(base, CHUNK)], ia_v)
        pltpu.sync_copy(ib_hbm.at[pl.ds(base, CHUNK)], ib_v)
        for i in range(CHUNK // 16):
            if offA:
                ia_v[pl.ds(i * 16, 16)] = ia_v[pl.ds(i * 16, 16)] + offA
            if offB:
                ib_v[pl.ds(i * 16, 16)] = ib_v[pl.ds(i * 16, 16)] + offB
        cpA = pltpu.async_copy(postab.at[ia_v], pa_v, semA)
        cpB = pltpu.async_copy(postab.at[ib_v], pb_v, semB)
        cpA.wait()
        cpB.wait()

        @pl.loop(0, CHUNK)
        def _(r):
            d16_v[r, pl.ds(0, 16)] = (pb_v[r, pl.ds(0, 16)]
                                      - pa_v[r, pl.ds(0, 16)])

        pltpu.sync_copy(d16_v, diff_hbm.at[pl.ds(base, CHUNK)])


def _prep_diff(offA, offB):
    return pl.kernel(
        functools.partial(_diff_body, offA, offB),
        out_type=jax.ShapeDtypeStruct((E_PAD, 16), jnp.float32),
        mesh=_mesh,
        scratch_types=[
            pltpu.VMEM((CHUNK,), jnp.int32),
            pltpu.VMEM((CHUNK,), jnp.int32),
            pltpu.VMEM((CHUNK, 128), jnp.float32),
            pltpu.VMEM((CHUNK, 128), jnp.float32),
            pltpu.VMEM((CHUNK, 16), jnp.float32),
            pltpu.SemaphoreType.DMA,
            pltpu.SemaphoreType.DMA,
        ],
    )


def _agg_body(two_src, sub, refs):
    if two_src:
        (srcA_hbm, srcB_hbm, dst_hbm, htabA, htabB, rw_hbm,
         msg_hbm, idxA_v, idxB_v, dst_v, rowsA_v, rowsB_v, rw_v,
         msg_v, semA, semB) = refs
    else:
        (srcA_hbm, dst_hbm, htabA, rw_hbm,
         msg_hbm, idxA_v, idxB_v, dst_v, rowsA_v, rowsB_v, rw_v,
         msg_v, semA, semB) = refs

    c = lax.axis_index("c")
    s = lax.axis_index("s")

    @pl.loop(0, ECH_PT)
    def _(k):
        base = s * (ECH_PT * CHUNK) + k * CHUNK
        pltpu.sync_copy(srcA_hbm.at[pl.ds(base, CHUNK)], idxA_v)
        cpA = pltpu.async_copy(htabA.at[idxA_v], rowsA_v, semA)
        if two_src:
            pltpu.sync_copy(srcB_hbm.at[pl.ds(base, CHUNK)], idxB_v)
            cpB = pltpu.async_copy(htabB.at[idxB_v], rowsB_v, semB)
        pltpu.sync_copy(rw_hbm.at[pl.ds(base, CHUNK)], rw_v)
        cpA.wait()
        if two_src:
            cpB.wait()
        for i in range(CHUNK):
            for o in (0, 16):
                m = (_half_row(rowsA_v, c, i, o)
                     * _half_row(rw_v, c, i, sub + o))
                if two_src:
                    m = m * _half_row(rowsB_v, c, i, o)
                msg_v[i, pl.ds(o, 16)] = m
        pltpu.sync_copy(msg_v, msg_hbm.at[pl.ds(c * E_PAD + base, CHUNK)])


def _edge_agg(two_src, sub):
    scratch = [
        pltpu.VMEM((CHUNK,), jnp.int32),
        pltpu.VMEM((CHUNK,), jnp.int32),
        pltpu.VMEM((CHUNK,), jnp.int32),
        pltpu.VMEM((CHUNK, 128), jnp.float32),
        pltpu.VMEM((CHUNK, 128), jnp.float32),
        pltpu.VMEM((CHUNK, 128), jnp.float32),
        pltpu.VMEM((CHUNK, HALF), jnp.float32),
        pltpu.SemaphoreType.DMA,
        pltpu.SemaphoreType.DMA,
    ]

    def body(*refs):
        _agg_body(two_src, sub, refs)

    return pl.kernel(body,
                     out_type=jax.ShapeDtypeStruct((NC * E_PAD, HALF),
                                                   jnp.float32),
                     mesh=_mesh, scratch_types=scratch)


def _vec_body(two_src, refs):
    if two_src:
        (srcA_hbm, srcB_hbm, dst_hbm, htabA, htabB, rwv_hbm,
         sv_hbm, idxA_v, idxB_v, dst_v, rowsA_v, rowsB_v, rwv_v,
         sv_v, semA, semB) = refs
    else:
        (srcA_hbm, dst_hbm, htabA, rwv_hbm,
         sv_hbm, idxA_v, idxB_v, dst_v, rowsA_v, rowsB_v, rwv_v,
         sv_v, semA, semB) = refs

    c = lax.axis_index("c")
    s = lax.axis_index("s")

    _zero_rows(sv_v, HALF)

    @pl.loop(0, ECH_PT)
    def _(k):
        base = s * (ECH_PT * CHUNK) + k * CHUNK
        pltpu.sync_copy(srcA_hbm.at[pl.ds(base, CHUNK)], idxA_v)
        cpA = pltpu.async_copy(htabA.at[idxA_v], rowsA_v, semA)
        if two_src:
            pltpu.sync_copy(srcB_hbm.at[pl.ds(base, CHUNK)], idxB_v)
            cpB = pltpu.async_copy(htabB.at[idxB_v], rowsB_v, semB)
        pltpu.sync_copy(rwv_hbm.at[pl.ds(base, CHUNK)], rwv_v)
        cpA.wait()
        if two_src:
            cpB.wait()
        for i in range(CHUNK):
            if two_src:
                t0 = (_half_row(rowsA_v, c, i, 0) * _half_row(rowsB_v, c, i, 0)
                      * _half_row(rwv_v, c, i, 0))
                t1 = (_half_row(rowsA_v, c, i, 16)
                      * _half_row(rowsB_v, c, i, 16)
                      * _half_row(rwv_v, c, i, 16))
            else:
                t0 = _half_row(rowsA_v, c, i, 0) * _half_row(rwv_v, c, i, 0)
                t1 = _half_row(rowsA_v, c, i, 16) * _half_row(rwv_v, c, i, 16)
            v = t0 + t1
            for sh in (8, 4, 2, 1):
                sv_v[i, pl.ds(0, 16)] = v
                v = v + sv_v[i, pl.ds(sh, 16)]
            sc = v[0]
            sv_v[i, pl.ds(0, 16)] = rwv_v[i, pl.ds(64, 16)] * sc
        pltpu.sync_copy(sv_v, sv_hbm.at[pl.ds(c * E_PAD + base, CHUNK)])


def _edge_vec(two_src):
    scratch = [
        pltpu.VMEM((CHUNK,), jnp.int32),
        pltpu.VMEM((CHUNK,), jnp.int32),
        pltpu.VMEM((CHUNK,), jnp.int32),
        pltpu.VMEM((CHUNK, 128), jnp.float32),
        pltpu.VMEM((CHUNK, 128), jnp.float32),
        pltpu.VMEM((CHUNK, 128), jnp.float32),
        pltpu.VMEM((CHUNK, HALF), jnp.float32),
        pltpu.SemaphoreType.DMA,
        pltpu.SemaphoreType.DMA,
    ]

    def body(*refs):
        _vec_body(two_src, refs)

    return pl.kernel(body,
                     out_type=jax.ShapeDtypeStruct((NC * E_PAD, HALF),
                                                   jnp.float32),
                     mesh=_mesh, scratch_types=scratch)


def _rw_body(with_unit, diff_ref, wr_ref, rw_ref):
    i = pl.program_id(0)
    blk = diff_ref.shape[0]
    diff = diff_ref[...]
    x, y, zz = diff[:, 0:1], diff[:, 1:2], diff[:, 2:3]
    d2 = x * x + y * y + zz * zz
    d = jnp.sqrt(d2 + 1e-8)
    row = i * blk + lax.broadcasted_iota(jnp.int32, (blk, 1), 0)
    mask = row < E
    centers = (lax.broadcasted_iota(jnp.int32, (1, R), 1).astype(jnp.float32)
               * (CUTOFF / (R - 1)))
    g = R / CUTOFF
    r = jnp.where(mask, jnp.exp(-g * (d - centers) ** 2), 0.0)
    out = jnp.dot(r, wr_ref[0], preferred_element_type=jnp.float32)
    if with_unit:
        unit = jnp.where(mask, diff / d, 0.0)
        col = lax.broadcasted_iota(jnp.int32, (blk, 128), 1)
        unit128 = jnp.pad(unit, ((0, 0), (64, 48)))
        out = jnp.where((col >= 64) & (col < 80), unit128, out)
    rw_ref[0] = out


def _rw_tc(diff16, wr_stack, with_unit):
    nl = wr_stack.shape[0]
    blk = 2048
    grid = (E_PAD // blk, nl)
    return pl.pallas_call(
        functools.partial(_rw_body, with_unit),
        grid=grid,
        in_specs=[
            pl.BlockSpec((blk, 16), lambda i, j: (i, 0)),
            pl.BlockSpec((1, R, 128), lambda i, j: (j, 0, 0)),
        ],
        out_specs=pl.BlockSpec((1, blk, 128), lambda i, j: (j, i, 0)),
        out_shape=jax.ShapeDtypeStruct((nl, E_PAD, 128), jnp.float32),
    )(diff16, wr_stack)


def _node_body(h_ref, agg_ref, wh_ref, out_ref):
    h = h_ref[...]
    up = jnp.tanh(jnp.dot(agg_ref[...], wh_ref[...],
                          preferred_element_type=jnp.float32))
    out_ref[...] = h + jnp.pad(up, ((0, 0), (0, 64)))


def _node_update(h, agg, wh):
    blk = 1024
    return pl.pallas_call(
        _node_body,
        grid=(N_PAD // blk,),
        in_specs=[
            pl.BlockSpec((blk, 128), lambda i: (i, 0)),
            pl.BlockSpec((blk, D), lambda i: (i, 0)),
            pl.BlockSpec((D, D), lambda i: (0, 0)),
        ],
        out_specs=pl.BlockSpec((blk, 128), lambda i: (i, 0)),
        out_shape=jax.ShapeDtypeStruct((N_PAD, 128), jnp.float32),
    )(h, agg, wh)


def _geom_body(h_ref, pn_ref, pca_ref, pc_ref, wg_ref, out_ref):
    v1 = pn_ref[...] - pca_ref[...]
    v2 = pc_ref[...] - pca_ref[...]
    n1 = jnp.sqrt(jnp.sum(v1[:, :3] * v1[:, :3], axis=1, keepdims=True) + 1e-8)
    n2 = jnp.sqrt(jnp.sum(v2[:, :3] * v2[:, :3], axis=1, keepdims=True) + 1e-8)
    cos = jnp.sum(v1[:, :3] * v2[:, :3], axis=1, keepdims=True) / (n1 * n2)
    geom = jnp.concatenate([n1, n2, cos, jnp.zeros_like(n1)], axis=1)
    up = jnp.tanh(jnp.dot(geom, wg_ref[...],
                          preferred_element_type=jnp.float32))
    out_ref[...] = h_ref[...] + jnp.pad(up, ((0, 0), (0, 64)))


def _geom_tc(h0, pn4, pca4, pc4, wg4):
    blk = 1024
    return pl.pallas_call(
        _geom_body,
        grid=(N_PAD // blk,),
        in_specs=[
            pl.BlockSpec((blk, 128), lambda i: (i, 0)),
            pl.BlockSpec((blk, 4), lambda i: (i, 0)),
            pl.BlockSpec((blk, 4), lambda i: (i, 0)),
            pl.BlockSpec((blk, 4), lambda i: (i, 0)),
            pl.BlockSpec((4, D), lambda i: (0, 0)),
        ],
        out_specs=pl.BlockSpec((blk, 128), lambda i: (i, 0)),
        out_shape=jax.ShapeDtypeStruct((N_PAD, 128), jnp.float32),
    )(h0, pn4, pca4, pc4, wg4)


def _final_body(v0_ref, v1_ref, v2_ref, v3_ref, hs_ref, ws_ref, noise_ref,
                mass_ref, gamma_ref, out_ref):
    vec = v0_ref[...] + v1_ref[...] + v2_ref[...] + v3_ref[...]
    h64 = hs_ref[...][:, :D]
    logits = jnp.dot(h64, ws_ref[...], preferred_element_type=jnp.float32)
    sigma = jnp.logaddexp(logits, 0.0)
    F = vec + gamma_ref[...] * noise_ref[...] * sigma
    out_ref[...] = F / mass_ref[...]


def _final_tc(vecs, h_sig, ws, noise, mass, gamma32):
    blk = 1024
    vecs3 = vecs
    return pl.pallas_call(
        _final_body,
        grid=(N_PAD // blk,),
        in_specs=[
            pl.BlockSpec((blk, HALF), lambda i: (i, 0)),
            pl.BlockSpec((blk, HALF), lambda i: (i, 0)),
            pl.BlockSpec((blk, HALF), lambda i: (i, 0)),
            pl.BlockSpec((blk, HALF), lambda i: (i, 0)),
            pl.BlockSpec((blk, 128), lambda i: (i, 0)),
            pl.BlockSpec((D, 1), lambda i: (0, 0)),
            pl.BlockSpec((blk, 1), lambda i: (i, 0)),
            pl.BlockSpec((blk, 1), lambda i: (i, 0)),
            pl.BlockSpec((1, HALF), lambda i: (0, 0)),
        ],
        out_specs=pl.BlockSpec((blk, HALF), lambda i: (i, 0)),
        out_shape=jax.ShapeDtypeStruct((N_PAD, HALF), jnp.float32),
    )(*vecs3, h_sig, ws, noise, mass, gamma32)


def _tab128(w):
    return jnp.pad(w, ((0, 0), (0, 64)))


def _pad_e(a, fill=0):
    return jnp.pad(a, (0, E_PAD - E), constant_values=fill)


def _pad128(p):
    return jnp.pad(p, ((0, 0), (0, 125)))


def kernel(t, velocity, ligand_positions, z, batch, ligand_mass, pos_N, pos_Ca,
           pos_C, residue_type, batch_residue, edge_index_ligand,
           edge_index_residue, edge_index_complex, params):
    pl_, ps_, pp_, pc_ = (params['lig'], params['sig'], params['prot'],
                          params['cpx'])

    tab = jnp.concatenate([_tab128(pl_['emb']), _tab128(ps_['emb']),
                           _tab128(pp_['res_emb'])], axis=0)
    z_pad = jnp.pad(z.astype(jnp.int32), (0, N_PAD - N))
    zr_pad = jnp.pad(residue_type.astype(jnp.int32), (0, N_PAD - N))
    postab = jnp.concatenate([_pad128(ligand_positions), _pad128(pos_Ca)], 0)

    srcl = _pad_e(edge_index_ligand[0].astype(jnp.int32))
    dstl = _pad_e(edge_index_ligand[1].astype(jnp.int32))
    srcr = _pad_e(edge_index_residue[0].astype(jnp.int32))
    dstr = _pad_e(edge_index_residue[1].astype(jnp.int32))
    ligc = _pad_e(edge_index_complex[0].astype(jnp.int32))
    resc = _pad_e(edge_index_complex[1].astype(jnp.int32))

    wr_lig = jnp.stack([jnp.concatenate([pl_['Wr0'], pl_['Wr1']], axis=1),
                        jnp.concatenate([ps_['Wr0'], ps_['Wr1']], axis=1)])
    wr_res = jnp.stack([jnp.pad(pp_['Wr'], ((0, 0), (0, 64)))])
    wr_cpx = jnp.stack([jnp.concatenate([pc_['Wr0'], pc_['Wr1']], axis=1)])
    wv_lig = jnp.stack([
        jnp.pad(pl_['Wr0'] * pl_['wv0'][None, :], ((0, 0), (0, 64))),
        jnp.pad(pl_['Wr1'] * pl_['wv1'][None, :], ((0, 0), (0, 64)))])
    wv_cpx = jnp.stack([
        jnp.pad(pc_['Wr0'] * pc_['wv0'][None, :], ((0, 0), (0, 64))),
        jnp.pad(pc_['Wr1'] * pc_['wv1'][None, :], ((0, 0), (0, 64)))])

    h_lig, h_sig, h_res = _prep_emb(tab, z_pad, zr_pad)
    diff_l = _prep_diff(0, 0)(srcl, dstl, postab)
    diff_r = _prep_diff(N, N)(srcr, dstr, postab)
    diff_c = _prep_diff(0, N)(ligc, resc, postab)

    rw_l = _rw_tc(diff_l, wr_lig, False)
    rw_r = _rw_tc(diff_r, wr_res, False)
    rw_c = _rw_tc(diff_c, wr_cpx, False)
    rwv_l = _rw_tc(diff_l, wv_lig, True)
    rwv_c = _rw_tc(diff_c, wv_cpx, True)

    wg4 = jnp.pad(pp_['Wg'], ((0, 1), (0, 0)))
    pad_n = ((0, N_PAD - N), (0, 0))
    pn4 = jnp.pad(jnp.pad(pos_N, ((0, 0), (0, 1))), pad_n)
    pca4 = jnp.pad(jnp.pad(pos_Ca, ((0, 0), (0, 1))), pad_n)
    pc4 = jnp.pad(jnp.pad(pos_C, ((0, 0), (0, 1))), pad_n)
    h_res = _geom_tc(h_res, pn4, pca4, pc4, wg4)

    aggA1 = _edge_agg(False, 0)
    aggB1 = _edge_agg(False, 64)
    aggA2 = _edge_agg(True, 0)
    vec1 = _edge_vec(False)
    vec2 = _edge_vec(True)

    def seg(rows, idx):
        m = rows.reshape(NC, E_PAD, HALF)
        return jax.ops.segment_sum(jnp.concatenate([m[0], m[1]], 1), idx,
                                   num_segments=N_PAD)

    def seg_sv(rows, idx):
        m = rows.reshape(NC, E_PAD, HALF)
        return jax.ops.segment_sum(m[0] + m[1], idx, num_segments=N_PAD)

    vecs = []
    h = h_lig
    for l in range(2):
        msg = (aggA1 if l == 0 else aggB1)(srcl, dstl, h, rw_l[0])
        vecs.append(seg_sv(vec1(srcl, dstl, h, rwv_l[l]), dstl))
        h = _node_update(h, seg(msg, dstl), pl_['Wh%d' % l])
    ligand_repr = h

    h = h_sig
    for l in range(2):
        msg = (aggA1 if l == 0 else aggB1)(srcl, dstl, h, rw_l[1])
        h = _node_update(h, seg(msg, dstl), ps_['Wh%d' % l])
    h_sig_f = h

    msg = aggA1(srcr, dstr, h_res, rw_r[0])
    h_res = _node_update(h_res, seg(msg, dstr), pp_['Wh'])

    h = ligand_repr
    for l in range(2):
        vecs.append(seg_sv(vec2(ligc, resc, ligc, h, h_res, rwv_c[l]), ligc))
        if l == 0:
            msg = aggA2(ligc, resc, ligc, h, h_res, rw_c[0])
            h = _node_update(h, seg(msg, ligc), pc_['Wh0'])

    noise = jax.random.normal(jax.random.key(42), (N, 1))
    noise = jnp.pad(noise, ((0, N_PAD - N), (0, 0)))
    mass = jnp.pad(ligand_mass.reshape(N, 1), ((0, N_PAD - N), (0, 0)),
                   constant_values=1.0)
    gamma32 = jnp.pad(params['gamma'], ((0, 0), (0, 29)))
    out = _final_tc(vecs, h_sig_f, params['w_sigma'], noise, mass, gamma32)
    return (out[:N, :3], velocity)
